# Initial kernel scaffold; baseline (speedup 1.0000x reference)
import jax
import jax.numpy as jnp
from jax.experimental import pallas as pl


def kernel(lidars):
    raise NotImplementedError("write your pallas kernel here")


# speedup vs baseline: 2.2377x; 2.2377x over previous
import jax
import jax.numpy as jnp
from jax import lax
from jax.experimental import pallas as pl
from jax.experimental.pallas import tpu as pltpu
from jax.experimental.pallas import tpu_sc as plsc

W = 400
H = 400
D = 12
T = 5
HW = H * W
DHW = D * HW
NPTS = T * 100000
GRID = T * DHW
PAD = 1024
PGRID = GRID + PAD

NTILES = 16
TILE_PTS = NPTS // NTILES
STAGE_PTS = 2048
NSTAGES = 16
GROUPS = STAGE_PTS // 16
ROW = 128
ROWS = NSTAGES * STAGE_PTS // ROW
ZCH = 9376
ZN = PGRID // NTILES // ZCH


def _body(pts, out, pbuf0, pbuf1, ibuf, zbuf, ones, zsem, psem, ssem):
    pbufs = (pbuf0, pbuf1)
    wid = lax.axis_index("s")
    iota = lax.iota(jnp.int32, 16)
    iota4 = iota * 4
    zvec = jnp.zeros((16,), jnp.float32)
    onev = jnp.full((16,), 1.0, jnp.float32)

    def _zb(i, c):
        zbuf[pl.ds(i * 16, 16)] = zvec
        return c

    lax.fori_loop(0, ZCH // 16, _zb, 0)
    for i in range(ROW // 16):
        ones[pl.ds(i * 16, 16)] = onev

    zbase = wid * (PGRID // NTILES)

    def _zfire(k, c):
        pltpu.async_copy(zbuf, out.at[pl.ds(zbase + k * ZCH, ZCH)], zsem)
        return c

    lax.fori_loop(0, ZN, _zfire, 0)

    base = wid * TILE_PTS

    def _sstart(s):
        return jnp.minimum(base + s * STAGE_PTS, NPTS - STAGE_PTS)

    def _pt_copy(s):
        return pltpu.make_async_copy(
            pts.at[pl.ds(_sstart(s) * 4, STAGE_PTS * 4)],
            pbufs[s % 2], psem)

    _pt_copy(0).start()
    for s in range(NSTAGES):
        if s + 1 < NSTAGES:
            _pt_copy(s + 1).start()
        _pt_copy(s).wait()
        pb = pbufs[s % 2]
        sp = _sstart(s)

        def _grp(g, c, s=s, pb=pb, sp=sp):
            ix = iota4 + g * 64
            xs = plsc.load_gather(pb, [ix])
            ys = plsc.load_gather(pb, [ix + 1])
            zs = plsc.load_gather(pb, [ix + 2])
            tw = (xs + 50.0) * 4.0
            th = (ys + 50.0) * 4.0
            td = (zs + 3.0) * 2.0
            iw = jnp.minimum(jnp.maximum(tw, -1.0), 512.0).astype(jnp.int32)
            ih = jnp.minimum(jnp.maximum(th, -1.0), 512.0).astype(jnp.int32)
            idd = jnp.minimum(jnp.maximum(td, -1.0), 64.0).astype(jnp.int32)
            valid = ((tw >= 0.0) & (th >= 0.0) & (td >= 0.0)
                     & (iw < W) & (ih < H) & (idd < D))
            pid = sp + g * 16 + iota
            tpl = (jnp.where(pid >= 100000, DHW, 0)
                   + jnp.where(pid >= 200000, DHW, 0)
                   + jnp.where(pid >= 300000, DHW, 0)
                   + jnp.where(pid >= 400000, DHW, 0))
            flat = tpl + idd * HW + ih * W + iw
            flat = jnp.where(valid, flat, GRID + wid * 64 + iota)
            row = s * (STAGE_PTS // ROW) + lax.shift_right_logical(g, 3)
            ibuf[row, pl.ds(lax.rem(g, 8) * 16, 16)] = flat
            return c

        lax.fori_loop(0, GROUPS, _grp, 0)

    def _zdrain(k, c):
        pltpu.make_async_copy(zbuf, out.at[pl.ds(zbase + k * ZCH, ZCH)],
                              zsem).wait()
        return c

    lax.fori_loop(0, ZN, _zdrain, 0)
    plsc.subcore_barrier()

    def _sfire(r, c):
        pltpu.async_copy(ones, out.at[ibuf.at[r]], ssem)
        return c

    lax.fori_loop(0, ROWS, _sfire, 0)

    def _sdrain(r, c):
        pltpu.make_async_copy(ones, out.at[ibuf.at[0]], ssem).wait()
        return c

    lax.fori_loop(0, ROWS, _sdrain, 0)


@jax.jit
def _voxelize(pts):
    mesh = plsc.VectorSubcoreMesh(
        core_axis_name="c", subcore_axis_name="s", num_cores=1)
    grid = pl.kernel(
        _body,
        out_type=jax.ShapeDtypeStruct((PGRID,), jnp.float32),
        mesh=mesh,
        compiler_params=pltpu.CompilerParams(needs_layout_passes=False),
        scratch_types=[
            pltpu.VMEM((STAGE_PTS * 4,), jnp.float32),
            pltpu.VMEM((STAGE_PTS * 4,), jnp.float32),
            pltpu.VMEM((ROWS, ROW), jnp.int32),
            pltpu.VMEM((ZCH,), jnp.float32),
            pltpu.VMEM((ROW,), jnp.float32),
            pltpu.SemaphoreType.DMA,
            pltpu.SemaphoreType.DMA,
            pltpu.SemaphoreType.DMA,
        ],
    )(pts)
    return grid[:GRID].reshape(T * D, H, W)


def kernel(lidars):
    return _voxelize(lidars.reshape(-1))


# speedup vs baseline: 4.7265x; 2.1122x over previous
import jax
import jax.numpy as jnp
from jax import lax
from jax.experimental import pallas as pl
from jax.experimental.pallas import tpu as pltpu
from jax.experimental.pallas import tpu_sc as plsc

W = 400
H = 400
D = 12
T = 5
HW = H * W
DHW = D * HW
NPTS = T * 100000
GRID = T * DHW
PAD = 1024
PGRID = GRID + PAD

NTILES = 16
TILE_PTS = 31256
STAGE_PTS = 2048
NSTAGES = 16
GROUPS = STAGE_PTS // 16
ROW = 128
ROWS = NSTAGES * STAGE_PTS // ROW
ZCH = 9376
ZN = PGRID // NTILES // ZCH


def _body(xs_h, ys_h, zs_h, out, pbx0, pbx1, pby0, pby1, pbz0, pbz1,
          ibuf, zbuf, ones, zsem, psem, ssem):
    pbx = (pbx0, pbx1)
    pby = (pby0, pby1)
    pbz = (pbz0, pbz1)
    wid = lax.axis_index("s")
    iota = lax.iota(jnp.int32, 16)
    zvec = jnp.zeros((16,), jnp.float32)
    onev = jnp.full((16,), 1.0, jnp.float32)

    def _zb(i, c):
        zbuf[pl.ds(i * 16, 16)] = zvec
        return c

    lax.fori_loop(0, ZCH // 16, _zb, 0)
    for i in range(ROW // 16):
        ones[pl.ds(i * 16, 16)] = onev

    zbase = wid * (PGRID // NTILES)

    def _zfire(k, c):
        pltpu.async_copy(zbuf, out.at[pl.ds(zbase + k * ZCH, ZCH)], zsem)
        return c

    lax.fori_loop(0, ZN, _zfire, 0)

    base = wid * TILE_PTS

    def _sstart(s):
        return jnp.minimum(base + s * STAGE_PTS, NPTS - STAGE_PTS)

    def _pt_copies(s):
        sl = pl.ds(_sstart(s) * 1, STAGE_PTS)
        b = s % 2
        return (pltpu.make_async_copy(xs_h.at[sl], pbx[b], psem),
                pltpu.make_async_copy(ys_h.at[sl], pby[b], psem),
                pltpu.make_async_copy(zs_h.at[sl], pbz[b], psem))

    for cp in _pt_copies(0):
        cp.start()
    for s in range(NSTAGES):
        if s + 1 < NSTAGES:
            for cp in _pt_copies(s + 1):
                cp.start()
        for cp in _pt_copies(s):
            cp.wait()
        bx, by, bz = pbx[s % 2], pby[s % 2], pbz[s % 2]
        sp = _sstart(s)

        def _grp(g, c, s=s, bx=bx, by=by, bz=bz, sp=sp):
            o = pl.ds(g * 16, 16)
            x = bx[o]
            y = by[o]
            z = bz[o]
            tw = (x + 50.0) * 4.0
            th = (y + 50.0) * 4.0
            td = (z + 3.0) * 2.0
            iw = jnp.minimum(jnp.maximum(tw, -1.0), 512.0).astype(jnp.int32)
            ih = jnp.minimum(jnp.maximum(th, -1.0), 512.0).astype(jnp.int32)
            idd = jnp.minimum(jnp.maximum(td, -1.0), 64.0).astype(jnp.int32)
            valid = ((tw >= 0.0) & (th >= 0.0) & (td >= 0.0)
                     & (iw < W) & (ih < H) & (idd < D))
            pid = sp + g * 16 + iota
            tpl = (jnp.where(pid >= 100000, DHW, 0)
                   + jnp.where(pid >= 200000, DHW, 0)
                   + jnp.where(pid >= 300000, DHW, 0)
                   + jnp.where(pid >= 400000, DHW, 0))
            flat = tpl + idd * HW + ih * W + iw
            flat = jnp.where(valid, flat, GRID + wid * 64 + iota)
            row = s * (STAGE_PTS // ROW) + lax.shift_right_logical(g, 3)
            ibuf[row, pl.ds(lax.rem(g, 8) * 16, 16)] = flat
            return c

        lax.fori_loop(0, GROUPS, _grp, 0)

    def _zdrain(k, c):
        pltpu.make_async_copy(zbuf, out.at[pl.ds(zbase + k * ZCH, ZCH)],
                              zsem).wait()
        return c

    lax.fori_loop(0, ZN, _zdrain, 0)
    plsc.subcore_barrier()

    def _sfire(r, c):
        pltpu.async_copy(ones, out.at[ibuf.at[r]], ssem)
        return c

    lax.fori_loop(0, ROWS, _sfire, 0)

    def _sdrain(r, c):
        pltpu.make_async_copy(ones, out.at[ibuf.at[0]], ssem).wait()
        return c

    lax.fori_loop(0, ROWS, _sdrain, 0)


@jax.jit
def _voxelize(xs, ys, zs):
    mesh = plsc.VectorSubcoreMesh(
        core_axis_name="c", subcore_axis_name="s", num_cores=1)
    grid = pl.kernel(
        _body,
        out_type=jax.ShapeDtypeStruct((PGRID,), jnp.float32),
        mesh=mesh,
        compiler_params=pltpu.CompilerParams(needs_layout_passes=False),
        scratch_types=(
            [pltpu.VMEM((STAGE_PTS,), jnp.float32) for _ in range(6)]
            + [
                pltpu.VMEM((ROWS, ROW), jnp.int32),
                pltpu.VMEM((ZCH,), jnp.float32),
                pltpu.VMEM((ROW,), jnp.float32),
                pltpu.SemaphoreType.DMA,
                pltpu.SemaphoreType.DMA,
                pltpu.SemaphoreType.DMA,
            ]
        ),
    )(xs, ys, zs)
    return jnp.maximum(grid[:GRID].reshape(T * D, H, W), 0.0)


def kernel(lidars):
    pts = lidars[0]
    xs = pts[:, :, 0].reshape(-1)
    ys = pts[:, :, 1].reshape(-1)
    zs = pts[:, :, 2].reshape(-1)
    return _voxelize(xs, ys, zs)
